# baseline (speedup 1.0000x reference)
import jax
import jax.numpy as jnp
from jax import lax
from jax.experimental import pallas as pl
from jax.experimental.pallas import tpu as pltpu
from jax.experimental.pallas import tpu_sc as plsc

NC = 2
NS = 16
NW = NC * NS
NBLK = 4
ROW = 8


def _gather_body(vocab_hbm, idx_hbm, out_hbm,
                 idx_v, rows0, rows1, gsem0, gsem1, wsem0, wsem1):
    wid = lax.axis_index("s") * NC + lax.axis_index("c")
    blk = idx_hbm.shape[1]
    rows = (rows0, rows1)
    gsems = (gsem0, gsem1)
    wsems = (wsem0, wsem1)

    pltpu.sync_copy(idx_hbm.at[pl.ds(wid * NBLK, NBLK)], idx_v)

    copies = [None, None]
    writes = [None, None]
    copies[0] = pltpu.async_copy(vocab_hbm.at[idx_v.at[0]], rows0, gsem0)
    for t in range(NBLK):
        b = t % 2
        nb = (t + 1) % 2
        if t + 1 < NBLK:
            if writes[nb] is not None:
                writes[nb].wait()
            copies[nb] = pltpu.async_copy(
                vocab_hbm.at[idx_v.at[t + 1]], rows[nb], gsems[nb])
        copies[b].wait()
        writes[b] = pltpu.async_copy(
            rows[b].at[:, pl.ds(0, 4)],
            out_hbm.at[pl.ds((wid * NBLK + t) * blk, blk)],
            wsems[b])
    writes[0].wait()
    writes[1].wait()


def kernel(batch, bs, seq_len, vocabulary):
    del bs, seq_len
    bs_static, seq_len_static = batch.shape
    tokens_per_item = vocabulary.shape[1]
    n = bs_static * seq_len_static
    blk = n // (NW * NBLK)
    vocab8 = jnp.pad(vocabulary, ((0, 0), (0, ROW - tokens_per_item)))
    idx_hbm = batch.reshape(NW * NBLK, blk)

    mesh = plsc.VectorSubcoreMesh(core_axis_name="c", subcore_axis_name="s")
    run = pl.kernel(
        _gather_body,
        out_type=jax.ShapeDtypeStruct((n, tokens_per_item), jnp.int32),
        mesh=mesh,
        scratch_types=[
            pltpu.VMEM((NBLK, blk), jnp.int32),
            pltpu.VMEM((blk, ROW), jnp.int32),
            pltpu.VMEM((blk, ROW), jnp.int32),
            pltpu.SemaphoreType.DMA,
            pltpu.SemaphoreType.DMA,
            pltpu.SemaphoreType.DMA,
            pltpu.SemaphoreType.DMA,
        ],
        compiler_params=pltpu.CompilerParams(use_tc_tiling_on_sc=False),
    )
    out = run(vocab8, idx_hbm)
    return out.reshape(bs_static, seq_len_static * tokens_per_item)


# speedup vs baseline: 1.6674x; 1.6674x over previous
import jax
import jax.numpy as jnp
from jax import lax
from jax.experimental import pallas as pl
from jax.experimental.pallas import tpu as pltpu
from jax.experimental.pallas import tpu_sc as plsc

NC = 2
NS = 16
NW = NC * NS
NBLK = 4
ROW = 8


def _gather_body(vocab_hbm, idx_hbm, out_hbm,
                 idx_v, rows0, rows1, gsem0, gsem1, wsem0, wsem1):
    wid = lax.axis_index("s") * NC + lax.axis_index("c")
    blk = idx_hbm.shape[1]
    rows = (rows0, rows1)
    gsems = (gsem0, gsem1)
    wsems = (wsem0, wsem1)

    pltpu.sync_copy(idx_hbm.at[pl.ds(wid * NBLK, NBLK)], idx_v)

    copies = [None, None]
    writes = [None, None]
    copies[0] = pltpu.async_copy(vocab_hbm.at[idx_v.at[0]], rows0, gsem0)
    for t in range(NBLK):
        b = t % 2
        nb = (t + 1) % 2
        if t + 1 < NBLK:
            if writes[nb] is not None:
                writes[nb].wait()
            copies[nb] = pltpu.async_copy(
                vocab_hbm.at[idx_v.at[t + 1]], rows[nb], gsems[nb])
        copies[b].wait()
        writes[b] = pltpu.async_copy(
            rows[b],
            out_hbm.at[pl.ds((wid * NBLK + t) * blk, blk)],
            wsems[b])
    writes[0].wait()
    writes[1].wait()


def kernel(batch, bs, seq_len, vocabulary):
    del bs, seq_len
    bs_static, seq_len_static = batch.shape
    tokens_per_item = vocabulary.shape[1]
    n = bs_static * seq_len_static
    blk = n // (NW * NBLK)
    vocab8 = jnp.pad(vocabulary, ((0, 0), (0, ROW - tokens_per_item)))
    idx_hbm = batch.reshape(NW * NBLK, blk)

    mesh = plsc.VectorSubcoreMesh(core_axis_name="c", subcore_axis_name="s")
    run = pl.kernel(
        _gather_body,
        out_type=jax.ShapeDtypeStruct((n, ROW), jnp.int32),
        mesh=mesh,
        scratch_types=[
            pltpu.VMEM((NBLK, blk), jnp.int32),
            pltpu.VMEM((blk, ROW), jnp.int32),
            pltpu.VMEM((blk, ROW), jnp.int32),
            pltpu.SemaphoreType.DMA,
            pltpu.SemaphoreType.DMA,
            pltpu.SemaphoreType.DMA,
            pltpu.SemaphoreType.DMA,
        ],
        compiler_params=pltpu.CompilerParams(use_tc_tiling_on_sc=False),
    )
    out = run(vocab8, idx_hbm)
    return out[:, :tokens_per_item].reshape(
        bs_static, seq_len_static * tokens_per_item)
